# Initial kernel scaffold; baseline (speedup 1.0000x reference)
import jax
import jax.numpy as jnp
from jax.experimental import pallas as pl


def kernel(x, pos_table):
    raise NotImplementedError("write your pallas kernel here")


# speedup vs baseline: 1.4602x; 1.4602x over previous
import jax
import jax.numpy as jnp
from jax.experimental import pallas as pl


def kernel(x, pos_table):
    B, S, E = x.shape
    BLK = 256

    def body(x_ref, p_ref, o_ref):
        o_ref[...] = x_ref[...] + p_ref[...]

    return pl.pallas_call(
        body,
        grid=(S // BLK, B),
        in_specs=[
            pl.BlockSpec((1, BLK, E), lambda s, b: (b, s, 0)),
            pl.BlockSpec((BLK, E), lambda s, b: (s, 0)),
        ],
        out_specs=pl.BlockSpec((1, BLK, E), lambda s, b: (b, s, 0)),
        out_shape=jax.ShapeDtypeStruct((B, S, E), x.dtype),
    )(x, pos_table)


# speedup vs baseline: 1.9200x; 1.3149x over previous
import jax
import jax.numpy as jnp
from jax.experimental import pallas as pl


def kernel(x, pos_table):
    B, S, E = x.shape
    BLK = 512

    def body(x_ref, p_ref, o_ref):
        o_ref[...] = x_ref[...] + p_ref[...]

    return pl.pallas_call(
        body,
        grid=(S // BLK, B),
        in_specs=[
            pl.BlockSpec((1, BLK, E), lambda s, b: (b, s, 0)),
            pl.BlockSpec((BLK, E), lambda s, b: (s, 0)),
        ],
        out_specs=pl.BlockSpec((1, BLK, E), lambda s, b: (b, s, 0)),
        out_shape=jax.ShapeDtypeStruct((B, S, E), x.dtype),
    )(x, pos_table)


# speedup vs baseline: 2.1127x; 1.1004x over previous
import jax
import jax.numpy as jnp
from jax.experimental import pallas as pl


def kernel(x, pos_table):
    B, S, E = x.shape
    BLK = 1024

    def body(x_ref, p_ref, o_ref):
        o_ref[...] = x_ref[...] + p_ref[...]

    return pl.pallas_call(
        body,
        grid=(S // BLK, B),
        in_specs=[
            pl.BlockSpec((1, BLK, E), lambda s, b: (b, s, 0)),
            pl.BlockSpec((BLK, E), lambda s, b: (s, 0)),
        ],
        out_specs=pl.BlockSpec((1, BLK, E), lambda s, b: (b, s, 0)),
        out_shape=jax.ShapeDtypeStruct((B, S, E), x.dtype),
    )(x, pos_table)


# speedup vs baseline: 2.2948x; 1.0862x over previous
import jax
import jax.numpy as jnp
from jax.experimental import pallas as pl


def kernel(x, pos_table):
    B, S, E = x.shape
    BLK = 2048

    def body(x_ref, p_ref, o_ref):
        o_ref[...] = x_ref[...] + p_ref[...]

    return pl.pallas_call(
        body,
        grid=(S // BLK, B),
        in_specs=[
            pl.BlockSpec((1, BLK, E), lambda s, b: (b, s, 0)),
            pl.BlockSpec((BLK, E), lambda s, b: (s, 0)),
        ],
        out_specs=pl.BlockSpec((1, BLK, E), lambda s, b: (b, s, 0)),
        out_shape=jax.ShapeDtypeStruct((B, S, E), x.dtype),
    )(x, pos_table)
